# baseline (speedup 1.0000x reference)
import functools

import jax
import jax.numpy as jnp
from jax import lax
from jax.experimental import pallas as pl
from jax.experimental.pallas import tpu as pltpu
from jax.experimental.pallas import tpu_sc as plsc

VOCAB = 1000
EMB = 128
BATCH = 4096
HIST = 50

B = BATCH * HIST
NC = 2
NS = 16
NW = NC * NS
BPW = B // NW
CH = 128
NCH = BPW // CH
NB = 6
GA = 3


def _embed_flat(idx3, W):
    mesh = plsc.VectorSubcoreMesh(core_axis_name="c", subcore_axis_name="s")

    @functools.partial(
        pl.kernel,
        mesh=mesh,
        out_type=jax.ShapeDtypeStruct((B, EMB), jnp.float32),
        scratch_types=[
            pltpu.VMEM((NCH, CH), jnp.int32),
            pltpu.VMEM((NB, CH, EMB), jnp.float32),
            pltpu.VMEM_SHARED((VOCAB, EMB), jnp.float32),
            pltpu.SemaphoreType.DMA((NB,)),
            pltpu.SemaphoreType.DMA((NB,)),
        ],
    )
    def k(table_hbm, idx_hbm, out_hbm, idx_v, bufs, tab_sh, gsem, ssem):
        cid = lax.axis_index("c")
        sid = lax.axis_index("s")
        wid = sid * NC + cid
        base = wid * BPW

        @pl.when(sid == 0)
        def _():
            pltpu.sync_copy(table_hbm, tab_sh)

        pltpu.sync_copy(idx_hbm.at[wid], idx_v)
        plsc.subcore_barrier()

        def g_start(j, b):
            pltpu.async_copy(tab_sh.at[idx_v.at[j]], bufs.at[b], gsem.at[b])

        def g_wait(j, b):
            pltpu.make_async_copy(
                tab_sh.at[idx_v.at[j]], bufs.at[b], gsem.at[b]
            ).wait()

        def s_start(j, b):
            dst = out_hbm.at[pl.ds(base + j * CH, CH)]
            pltpu.async_copy(bufs.at[b], dst, ssem.at[b])

        def s_wait(j, b):
            dst = out_hbm.at[pl.ds(base + j * CH, CH)]
            pltpu.make_async_copy(bufs.at[b], dst, ssem.at[b]).wait()

        def step(j, q, ahead, reuse):
            g_wait(j, q)
            s_start(j, q)
            if ahead:
                jj = j + GA
                qq = (q + GA) % NB
                if reuse:
                    s_wait(jj - NB, qq)
                g_start(jj, qq)

        for j in range(GA):
            g_start(j, j)
        for j in range(NB):
            step(j, j, True, j + GA >= NB)

        NPF = (NCH - NB) // NB
        def body(p, carry):
            for q in range(NB):
                step(p * NB + q, q, True, True)
            return carry

        lax.fori_loop(1, NPF, body, 0)

        for t in range(NCH - NPF * NB):
            j = NPF * NB + t
            step(j, j % NB, j + GA < NCH, True)

        for j in range(NCH - NB, NCH):
            s_wait(j, j % NB)

    return k(W, idx3)


def kernel(input, W):
    idx3 = input.T.reshape(NW, NCH, CH)
    out = _embed_flat(idx3, W)
    return out.reshape(HIST, BATCH, EMB).transpose(1, 0, 2)


# speedup vs baseline: 1.0064x; 1.0064x over previous
import functools

import jax
import jax.numpy as jnp
from jax import lax
from jax.experimental import pallas as pl
from jax.experimental.pallas import tpu as pltpu
from jax.experimental.pallas import tpu_sc as plsc

VOCAB = 1000
EMB = 128
BATCH = 4096
HIST = 50

B = BATCH * HIST
NC = 2
NS = 16
NW = NC * NS
BPW = B // NW
CH = 128
NCH = BPW // CH
NB = 4
NP = NCH // NB
TAIL = NCH - NP * NB


def _embed_flat(idx3, W):
    mesh = plsc.VectorSubcoreMesh(core_axis_name="c", subcore_axis_name="s")

    @functools.partial(
        pl.kernel,
        mesh=mesh,
        out_type=jax.ShapeDtypeStruct((B, EMB), jnp.float32),
        scratch_types=[
            pltpu.VMEM((NCH, CH), jnp.int32),
            pltpu.VMEM((NB, CH, EMB), jnp.float32),
            pltpu.VMEM_SHARED((VOCAB, EMB), jnp.float32),
            pltpu.SemaphoreType.DMA((NB,)),
            pltpu.SemaphoreType.DMA((NB,)),
        ],
    )
    def k(table_hbm, idx_hbm, out_hbm, idx_v, bufs, tab_sh, gsem, ssem):
        cid = lax.axis_index("c")
        sid = lax.axis_index("s")
        wid = sid * NC + cid
        base = wid * BPW

        @pl.when(sid == 0)
        def _():
            pltpu.sync_copy(table_hbm, tab_sh)

        pltpu.sync_copy(idx_hbm.at[wid], idx_v)
        plsc.subcore_barrier()

        def gather(j, b):
            pltpu.async_copy(tab_sh.at[idx_v.at[j]], bufs.at[b], gsem.at[b])

        def store(j, b):
            dst = out_hbm.at[pl.ds(base + j * CH, CH)]
            pltpu.async_copy(bufs.at[b], dst, ssem.at[b])
            return dst

        for b in range(NB):
            gather(b, b)

        def body(p, carry):
            for b in range(NB):
                j = p * NB + b
                pltpu.make_async_copy(
                    tab_sh.at[idx_v.at[j]], bufs.at[b], gsem.at[b]
                ).wait()
                dst = store(j, b)
                pltpu.make_async_copy(bufs.at[b], dst, ssem.at[b]).wait()
                gather(j + NB, b)
            return carry

        lax.fori_loop(0, NP - 1, body, 0)

        for t in range(NB + TAIL):
            j = (NP - 1) * NB + t
            b = t % NB
            pltpu.make_async_copy(
                tab_sh.at[idx_v.at[j]], bufs.at[b], gsem.at[b]
            ).wait()
            dst = store(j, b)
            pltpu.make_async_copy(bufs.at[b], dst, ssem.at[b]).wait()
            if t + NB < NB + TAIL:
                gather(j + NB, b)

    return k(W, idx3)


def kernel(input, W):
    idx3 = input.T.reshape(NW, NCH, CH)
    out = _embed_flat(idx3, W)
    return out.reshape(HIST, BATCH, EMB).transpose(1, 0, 2)


# speedup vs baseline: 1.0078x; 1.0014x over previous
import functools

import jax
import jax.numpy as jnp
from jax import lax
from jax.experimental import pallas as pl
from jax.experimental.pallas import tpu as pltpu
from jax.experimental.pallas import tpu_sc as plsc

VOCAB = 1000
EMB = 128
BATCH = 4096
HIST = 50

B = BATCH * HIST
NC = 2
NS = 16
NW = NC * NS
CH = BATCH // NW
NCH = HIST
NB = 4
NP = NCH // NB
TAIL = NCH - NP * NB


def _embed_flat(idxT, W):
    mesh = plsc.VectorSubcoreMesh(core_axis_name="c", subcore_axis_name="s")

    @functools.partial(
        pl.kernel,
        mesh=mesh,
        out_type=jax.ShapeDtypeStruct((B, EMB), jnp.float32),
        scratch_types=[
            pltpu.VMEM((NCH, CH), jnp.int32),
            pltpu.VMEM((NB, CH, EMB), jnp.float32),
            pltpu.VMEM_SHARED((VOCAB, EMB), jnp.float32),
            pltpu.SemaphoreType.DMA((NB,)),
            pltpu.SemaphoreType.DMA((NB,)),
        ],
    )
    def k(table_hbm, idx_hbm, out_hbm, idx_v, bufs, tab_sh, gsem, ssem):
        cid = lax.axis_index("c")
        sid = lax.axis_index("s")
        wid = sid * NC + cid
        col = wid * CH

        @pl.when(sid == 0)
        def _():
            pltpu.sync_copy(table_hbm, tab_sh)

        pltpu.sync_copy(idx_hbm.at[:, pl.ds(col, CH)], idx_v)
        plsc.subcore_barrier()

        def gather(j, b):
            pltpu.async_copy(tab_sh.at[idx_v.at[j]], bufs.at[b], gsem.at[b])

        def store(j, b):
            dst = out_hbm.at[pl.ds(j * BATCH + col, CH)]
            pltpu.async_copy(bufs.at[b], dst, ssem.at[b])
            return dst

        for b in range(NB):
            gather(b, b)

        def body(p, carry):
            for b in range(NB):
                j = p * NB + b
                pltpu.make_async_copy(
                    tab_sh.at[idx_v.at[j]], bufs.at[b], gsem.at[b]
                ).wait()
                dst = store(j, b)
                pltpu.make_async_copy(bufs.at[b], dst, ssem.at[b]).wait()
                gather(j + NB, b)
            return carry

        lax.fori_loop(0, NP - 1, body, 0)

        for t in range(NB + TAIL):
            j = (NP - 1) * NB + t
            b = t % NB
            pltpu.make_async_copy(
                tab_sh.at[idx_v.at[j]], bufs.at[b], gsem.at[b]
            ).wait()
            dst = store(j, b)
            pltpu.make_async_copy(bufs.at[b], dst, ssem.at[b]).wait()
            if t + NB < NB + TAIL:
                gather(j + NB, b)

    return k(W, idxT)


def kernel(input, W):
    out = _embed_flat(input.T, W)
    return out.reshape(HIST, BATCH, EMB).transpose(1, 0, 2)
